# baseline (speedup 1.0000x reference)
import functools

import jax
import jax.numpy as jnp
from jax import lax
from jax.experimental import pallas as pl
from jax.experimental.pallas import tpu as pltpu
from jax.experimental.pallas import tpu_sc as plsc

N = 10000
E = 320000
D_IN = 128
HID = 64
D_OUT = 128

NC = 2
NS = 16
NWORK = NC * NS
EPW = E // NWORK
WIN = 125
WPS = EPW // WIN
NBUF = 5
WMAIN = ((WPS - NBUF) // NBUF) * NBUF
N_PAD = 10240
RPS = N_PAD // NS
DEGW = 16
NH = N_PAD // 2
AP = NC * N_PAD * HID // 128
NER = E // 128

_mesh = plsc.VectorSubcoreMesh(core_axis_name="c", subcore_axis_name="s")
_sc_params = pltpu.CompilerParams(use_tc_tiling_on_sc=False)


DPS = NH // NS


@functools.partial(
    pl.kernel,
    out_type=jax.ShapeDtypeStruct((NC * NH, 128), jnp.float32),
    mesh=_mesh,
    scratch_types=[
        pltpu.VMEM((WPS, WIN), jnp.int32),
        pltpu.VMEM((WIN, DEGW), jnp.float32),
        pltpu.VMEM_SHARED((N_PAD, DEGW), jnp.float32),
        pltpu.VMEM((DPS, DEGW), jnp.float32),
        pltpu.VMEM((DPS, DEGW), jnp.float32),
        pltpu.VMEM((DPS, 128), jnp.float32),
    ],
    compiler_params=_sc_params,
)
def _deg_kernel(ei_hbm, vfull_hbm, zeros_hbm, out_hbm, dst_v, vfull, acc,
                st_lo, st_hi, st_out):
    c = lax.axis_index("c")
    s = lax.axis_index("s")
    wid = c * NS + s
    r0 = s * RPS
    pltpu.sync_copy(zeros_hbm.at[pl.ds(r0, RPS)], acc.at[pl.ds(r0, RPS)])
    pltpu.sync_copy(ei_hbm.at[1, wid], dst_v)
    pltpu.sync_copy(vfull_hbm, vfull)
    plsc.subcore_barrier()

    @pl.loop(0, WPS)
    def _(w):
        pltpu.sync_copy(vfull, acc.at[dst_v.at[w]], add=True)

    plsc.subcore_barrier()
    pltpu.sync_copy(acc.at[pl.ds(s * DPS, DPS)], st_lo)
    pltpu.sync_copy(acc.at[pl.ds(NH + s * DPS, DPS)], st_hi)

    @pl.loop(0, DPS)
    def _(r):
        lo = st_lo.at[r][...]
        hi = st_hi.at[r][...]
        for k in range(4):
            st_out.at[r, pl.ds(16 * k, 16)][...] = lo
            st_out.at[r, pl.ds(64 + 16 * k, 16)][...] = hi

    pltpu.sync_copy(st_out, out_hbm.at[pl.ds(c * NH + s * DPS, DPS)])


@functools.partial(
    pl.kernel,
    out_type=jax.ShapeDtypeStruct((NC * N_PAD, HID), jnp.float32),
    mesh=_mesh,
    scratch_types=[
        pltpu.VMEM((WPS, WIN), jnp.int32),
        pltpu.VMEM((WPS, WIN), jnp.int32),
    ] + [pltpu.VMEM((WIN, HID), jnp.float32) for _ in range(NBUF)] + [
        pltpu.VMEM_SHARED((N_PAD, HID), jnp.float32),
    ] + [pltpu.SemaphoreType.DMA for _ in range(NBUF)],
    compiler_params=_sc_params,
)
def _msg_kernel(hs_hbm, srcw_hbm, dstw_hbm, zeros_hbm, out_hbm,
                src_v, dst_v, *rest):
    bufs = rest[:NBUF]
    acc = rest[NBUF]
    gsem = rest[NBUF + 1:]
    c = lax.axis_index("c")
    s = lax.axis_index("s")
    wid = c * NS + s
    r0 = s * RPS
    pltpu.sync_copy(zeros_hbm.at[pl.ds(r0, RPS)], acc.at[pl.ds(r0, RPS)])
    pltpu.sync_copy(srcw_hbm.at[wid], src_v)
    pltpu.sync_copy(dstw_hbm.at[wid], dst_v)
    plsc.subcore_barrier()

    def _wait_gather(i):
        pltpu.make_async_copy(hs_hbm.at[pl.ds(0, WIN)], bufs[i], gsem[i]).wait()

    def _slot(w, i, issue):
        _wait_gather(i)
        pltpu.sync_copy(bufs[i], acc.at[dst_v.at[w]], add=True)
        if issue:
            pltpu.async_copy(hs_hbm.at[src_v.at[w + NBUF]], bufs[i], gsem[i])

    for i in range(NBUF):
        pltpu.async_copy(hs_hbm.at[src_v.at[i]], bufs[i], gsem[i])

    @pl.loop(0, WMAIN, step=NBUF)
    def _(w):
        for i in range(NBUF):
            _wait_gather(i)
            pltpu.sync_copy(bufs[i], acc.at[dst_v.at[w + i]], add=True)
            pltpu.async_copy(hs_hbm.at[src_v.at[w + NBUF + i]], bufs[i],
                             gsem[i])

    for w in range(WMAIN, WPS):
        _slot(w, w % NBUF, w + NBUF < WPS)
    plsc.subcore_barrier()
    pltpu.sync_copy(acc.at[pl.ds(r0, RPS)],
                    out_hbm.at[pl.ds(c * N_PAD + r0, RPS)])


def _t0_body(x_ref, w1_ref, ei_ref, hp_ref, srct_ref, dstt_ref):
    h = jnp.dot(x_ref[...], w1_ref[...], preferred_element_type=jnp.float32)
    lo = h[0:NH, :]
    hi = jnp.concatenate(
        [h[NH:N, :], jnp.zeros((N_PAD - N, HID), jnp.float32)], axis=0)
    hp_ref[...] = jnp.concatenate([lo, hi], axis=1)
    s = ei_ref[0]
    srct_ref[...] = 2 * s - (2 * NH - 1) * (s >= NH).astype(jnp.int32)
    d = ei_ref[1]
    dstt_ref[...] = 2 * d - (2 * NH - 1) * (d >= NH).astype(jnp.int32)


_TB = 640
_TG = NH // _TB


def _t1_body(hp_ref, dga_ref, dgb_ref, hsp_ref, dinvp_ref):
    dinv_p = lax.rsqrt(dga_ref[...] + dgb_ref[...] + 1.0)
    dinvp_ref[...] = dinv_p
    hsp_ref[...] = hp_ref[...] * dinv_p


def _t2_body(aga_ref, agb_ref, hsp_ref, dinvp_ref, b1p_ref, h2sp_ref):
    dinv_p = dinvp_ref[...]
    out1 = dinv_p * (aga_ref[...] + agb_ref[...] + hsp_ref[...]) + b1p_ref[...]
    r = jnp.maximum(out1, 0.0)
    s = r * r
    i0 = 1.0 / jnp.maximum(
        jnp.sqrt(jnp.sum(s[:, 0:HID], axis=1, keepdims=True)), 1e-12)
    i1 = 1.0 / jnp.maximum(
        jnp.sqrt(jnp.sum(s[:, HID:128], axis=1, keepdims=True)), 1e-12)
    scale = jnp.concatenate([jnp.broadcast_to(i0, (_TB, HID)),
                             jnp.broadcast_to(i1, (_TB, HID))], axis=1)
    h2sp_ref[...] = r * scale * dinv_p


def _t3_body(aggp_ref, h2sp_ref, dinvp_ref, w2_ref, b2_ref, out_ref):
    a0 = aggp_ref[0:NH, :]
    a1 = aggp_ref[NH:2 * NH, :]
    pre = dinvp_ref[...] * (a0 + a1 + h2sp_ref[...])
    o_lo = jnp.dot(pre[:, 0:HID], w2_ref[...],
                   preferred_element_type=jnp.float32)
    o_hi = jnp.dot(pre[:, HID:128], w2_ref[...],
                   preferred_element_type=jnp.float32)
    out_ref[...] = (
        jnp.concatenate([o_lo, o_hi[0:N - NH, :]], axis=0) + b2_ref[...])


_f32 = jnp.float32


def kernel(x, edge_index, W1, b1, W2, b2):
    ei_r = edge_index.reshape(2, NER, 128)
    ei_sc = ei_r.reshape(2, NWORK, WPS, WIN)
    zeros_deg = jnp.zeros((N_PAD, DEGW), _f32)
    zeros_hid = jnp.zeros((N_PAD, HID), _f32)
    vfull = jnp.ones((WIN, DEGW), _f32)
    b1p = jnp.concatenate([b1, b1]).reshape(1, 128)

    degp2 = _deg_kernel(ei_sc, vfull, zeros_deg)
    hp, srct, dstt = pl.pallas_call(
        _t0_body,
        out_shape=(jax.ShapeDtypeStruct((NH, 128), _f32),
                   jax.ShapeDtypeStruct((NER, 128), jnp.int32),
                   jax.ShapeDtypeStruct((NER, 128), jnp.int32)),
    )(x, W1, ei_r)
    srcw = srct.reshape(NWORK, WPS, WIN)
    dstw = dstt.reshape(NWORK, WPS, WIN)

    blk = pl.BlockSpec((_TB, 128), lambda i: (i, 0))
    blk_hi = pl.BlockSpec((_TB, 128), lambda i: (i + _TG, 0))
    hsp, dinvp = pl.pallas_call(
        _t1_body,
        grid=(_TG,),
        in_specs=[blk, blk, blk_hi],
        out_specs=[blk, blk],
        out_shape=(jax.ShapeDtypeStruct((NH, 128), _f32),
                   jax.ShapeDtypeStruct((NH, 128), _f32)),
    )(hp, degp2, degp2)

    agg1p = _msg_kernel(hsp.reshape(N_PAD, HID), srcw, dstw,
                        zeros_hid).reshape(AP, 128)

    h2sp = pl.pallas_call(
        _t2_body,
        grid=(_TG,),
        in_specs=[blk, blk_hi, blk, blk,
                  pl.BlockSpec((1, 128), lambda i: (0, 0))],
        out_specs=blk,
        out_shape=jax.ShapeDtypeStruct((NH, 128), _f32),
    )(agg1p, agg1p, hsp, dinvp, b1p)

    agg2p = _msg_kernel(h2sp.reshape(N_PAD, HID), srcw, dstw,
                        zeros_hid).reshape(AP, 128)

    out = pl.pallas_call(
        _t3_body,
        out_shape=jax.ShapeDtypeStruct((N, D_OUT), _f32),
    )(agg2p, h2sp, dinvp, W2, b2.reshape(1, D_OUT))

    return out


# speedup vs baseline: 1.0006x; 1.0006x over previous
import functools

import jax
import jax.numpy as jnp
from jax import lax
from jax.experimental import pallas as pl
from jax.experimental.pallas import tpu as pltpu
from jax.experimental.pallas import tpu_sc as plsc

N = 10000
E = 320000
D_IN = 128
HID = 64
D_OUT = 128

NC = 2
NS = 16
NWORK = NC * NS
EPW = E // NWORK
WIN = 125
WPS = EPW // WIN
NBUF = 4
WMAIN = ((WPS - NBUF) // NBUF) * NBUF
N_PAD = 10240
RPS = N_PAD // NS
DEGW = 16
NH = N_PAD // 2
AP = NC * N_PAD * HID // 128
NER = E // 128

_mesh = plsc.VectorSubcoreMesh(core_axis_name="c", subcore_axis_name="s")
_sc_params = pltpu.CompilerParams(use_tc_tiling_on_sc=False)


DPS = NH // NS


@functools.partial(
    pl.kernel,
    out_type=jax.ShapeDtypeStruct((NC * NH, 128), jnp.float32),
    mesh=_mesh,
    scratch_types=[
        pltpu.VMEM((WPS, WIN), jnp.int32),
        pltpu.VMEM((WIN, DEGW), jnp.float32),
        pltpu.VMEM_SHARED((N_PAD, DEGW), jnp.float32),
        pltpu.VMEM((DPS, DEGW), jnp.float32),
        pltpu.VMEM((DPS, DEGW), jnp.float32),
        pltpu.VMEM((DPS, 128), jnp.float32),
    ],
    compiler_params=_sc_params,
)
def _deg_kernel(ei_hbm, vfull_hbm, zeros_hbm, out_hbm, dst_v, vfull, acc,
                st_lo, st_hi, st_out):
    c = lax.axis_index("c")
    s = lax.axis_index("s")
    wid = c * NS + s
    r0 = s * RPS
    pltpu.sync_copy(zeros_hbm.at[pl.ds(r0, RPS)], acc.at[pl.ds(r0, RPS)])
    pltpu.sync_copy(ei_hbm.at[1, wid], dst_v)
    pltpu.sync_copy(vfull_hbm, vfull)
    plsc.subcore_barrier()

    @pl.loop(0, WPS)
    def _(w):
        pltpu.sync_copy(vfull, acc.at[dst_v.at[w]], add=True)

    plsc.subcore_barrier()
    pltpu.sync_copy(acc.at[pl.ds(s * DPS, DPS)], st_lo)
    pltpu.sync_copy(acc.at[pl.ds(NH + s * DPS, DPS)], st_hi)

    @pl.loop(0, DPS)
    def _(r):
        lo = st_lo.at[r][...]
        hi = st_hi.at[r][...]
        for k in range(4):
            st_out.at[r, pl.ds(16 * k, 16)][...] = lo
            st_out.at[r, pl.ds(64 + 16 * k, 16)][...] = hi

    pltpu.sync_copy(st_out, out_hbm.at[pl.ds(c * NH + s * DPS, DPS)])


@functools.partial(
    pl.kernel,
    out_type=jax.ShapeDtypeStruct((NC * N_PAD, HID), jnp.float32),
    mesh=_mesh,
    scratch_types=[
        pltpu.VMEM((WPS, WIN), jnp.int32),
        pltpu.VMEM((WPS, WIN), jnp.int32),
    ] + [pltpu.VMEM((WIN, HID), jnp.float32) for _ in range(NBUF)] + [
        pltpu.VMEM_SHARED((N_PAD, HID), jnp.float32),
    ] + [pltpu.SemaphoreType.DMA for _ in range(NBUF)],
    compiler_params=_sc_params,
)
def _msg_kernel(hs_hbm, srcw_hbm, dstw_hbm, zeros_hbm, out_hbm,
                src_v, dst_v, *rest):
    bufs = rest[:NBUF]
    acc = rest[NBUF]
    gsem = rest[NBUF + 1:]
    c = lax.axis_index("c")
    s = lax.axis_index("s")
    wid = c * NS + s
    r0 = s * RPS
    pltpu.sync_copy(zeros_hbm.at[pl.ds(r0, RPS)], acc.at[pl.ds(r0, RPS)])
    pltpu.sync_copy(srcw_hbm.at[wid], src_v)
    pltpu.sync_copy(dstw_hbm.at[wid], dst_v)
    plsc.subcore_barrier()

    def _wait_gather(i):
        pltpu.make_async_copy(hs_hbm.at[pl.ds(0, WIN)], bufs[i], gsem[i]).wait()

    def _slot(w, i, issue):
        _wait_gather(i)
        pltpu.sync_copy(bufs[i], acc.at[dst_v.at[w]], add=True)
        if issue:
            pltpu.async_copy(hs_hbm.at[src_v.at[w + NBUF]], bufs[i], gsem[i])

    for i in range(NBUF):
        pltpu.async_copy(hs_hbm.at[src_v.at[i]], bufs[i], gsem[i])

    @pl.loop(0, WMAIN, step=NBUF)
    def _(w):
        for i in range(NBUF):
            _wait_gather(i)
            pltpu.sync_copy(bufs[i], acc.at[dst_v.at[w + i]], add=True)
            pltpu.async_copy(hs_hbm.at[src_v.at[w + NBUF + i]], bufs[i],
                             gsem[i])

    for w in range(WMAIN, WPS):
        _slot(w, w % NBUF, w + NBUF < WPS)
    plsc.subcore_barrier()
    pltpu.sync_copy(acc.at[pl.ds(r0, RPS)],
                    out_hbm.at[pl.ds(c * N_PAD + r0, RPS)])


def _t0_body(x_ref, w1_ref, ei_ref, hp_ref, srct_ref, dstt_ref):
    h = jnp.dot(x_ref[...], w1_ref[...], preferred_element_type=jnp.float32)
    lo = h[0:NH, :]
    hi = jnp.concatenate(
        [h[NH:N, :], jnp.zeros((N_PAD - N, HID), jnp.float32)], axis=0)
    hp_ref[...] = jnp.concatenate([lo, hi], axis=1)
    s = ei_ref[0]
    srct_ref[...] = 2 * s - (2 * NH - 1) * (s >= NH).astype(jnp.int32)
    d = ei_ref[1]
    dstt_ref[...] = 2 * d - (2 * NH - 1) * (d >= NH).astype(jnp.int32)


_TB = 640
_TG = NH // _TB


def _t1_body(hp_ref, dga_ref, dgb_ref, hsp_ref, dinvp_ref):
    dinv_p = lax.rsqrt(dga_ref[...] + dgb_ref[...] + 1.0)
    dinvp_ref[...] = dinv_p
    hsp_ref[...] = hp_ref[...] * dinv_p


def _t2_body(aga_ref, agb_ref, hsp_ref, dinvp_ref, b1p_ref, h2sp_ref):
    dinv_p = dinvp_ref[...]
    out1 = dinv_p * (aga_ref[...] + agb_ref[...] + hsp_ref[...]) + b1p_ref[...]
    r = jnp.maximum(out1, 0.0)
    s = r * r
    i0 = 1.0 / jnp.maximum(
        jnp.sqrt(jnp.sum(s[:, 0:HID], axis=1, keepdims=True)), 1e-12)
    i1 = 1.0 / jnp.maximum(
        jnp.sqrt(jnp.sum(s[:, HID:128], axis=1, keepdims=True)), 1e-12)
    scale = jnp.concatenate([jnp.broadcast_to(i0, (_TB, HID)),
                             jnp.broadcast_to(i1, (_TB, HID))], axis=1)
    h2sp_ref[...] = r * scale * dinv_p


def _t3_body(aggp_ref, h2sp_ref, dinvp_ref, w2_ref, b2_ref, out_ref):
    a0 = aggp_ref[0:NH, :]
    a1 = aggp_ref[NH:2 * NH, :]
    pre = dinvp_ref[...] * (a0 + a1 + h2sp_ref[...])
    o_lo = jnp.dot(pre[:, 0:HID], w2_ref[...],
                   preferred_element_type=jnp.float32)
    o_hi = jnp.dot(pre[:, HID:128], w2_ref[...],
                   preferred_element_type=jnp.float32)
    out_ref[...] = (
        jnp.concatenate([o_lo, o_hi[0:N - NH, :]], axis=0) + b2_ref[...])


_f32 = jnp.float32


def kernel(x, edge_index, W1, b1, W2, b2):
    ei_r = edge_index.reshape(2, NER, 128)
    ei_sc = ei_r.reshape(2, NWORK, WPS, WIN)
    zeros_deg = jnp.zeros((N_PAD, DEGW), _f32)
    zeros_hid = jnp.zeros((N_PAD, HID), _f32)
    vfull = jnp.ones((WIN, DEGW), _f32)
    b1p = jnp.concatenate([b1, b1]).reshape(1, 128)

    degp2 = _deg_kernel(ei_sc, vfull, zeros_deg)
    hp, srct, dstt = pl.pallas_call(
        _t0_body,
        out_shape=(jax.ShapeDtypeStruct((NH, 128), _f32),
                   jax.ShapeDtypeStruct((NER, 128), jnp.int32),
                   jax.ShapeDtypeStruct((NER, 128), jnp.int32)),
    )(x, W1, ei_r)
    srcw = srct.reshape(NWORK, WPS, WIN)
    dstw = dstt.reshape(NWORK, WPS, WIN)

    blk = pl.BlockSpec((_TB, 128), lambda i: (i, 0))
    blk_hi = pl.BlockSpec((_TB, 128), lambda i: (i + _TG, 0))
    hsp, dinvp = pl.pallas_call(
        _t1_body,
        grid=(_TG,),
        in_specs=[blk, blk, blk_hi],
        out_specs=[blk, blk],
        out_shape=(jax.ShapeDtypeStruct((NH, 128), _f32),
                   jax.ShapeDtypeStruct((NH, 128), _f32)),
    )(hp, degp2, degp2)

    agg1p = _msg_kernel(hsp.reshape(N_PAD, HID), srcw, dstw,
                        zeros_hid).reshape(AP, 128)

    h2sp = pl.pallas_call(
        _t2_body,
        grid=(_TG,),
        in_specs=[blk, blk_hi, blk, blk,
                  pl.BlockSpec((1, 128), lambda i: (0, 0))],
        out_specs=blk,
        out_shape=jax.ShapeDtypeStruct((NH, 128), _f32),
    )(agg1p, agg1p, hsp, dinvp, b1p)

    agg2p = _msg_kernel(h2sp.reshape(N_PAD, HID), srcw, dstw,
                        zeros_hid).reshape(AP, 128)

    out = pl.pallas_call(
        _t3_body,
        out_shape=jax.ShapeDtypeStruct((N, D_OUT), _f32),
    )(agg2p, h2sp, dinvp, W2, b2.reshape(1, D_OUT))

    return out


# speedup vs baseline: 1.0191x; 1.0184x over previous
import functools

import jax
import jax.numpy as jnp
from jax import lax
from jax.experimental import pallas as pl
from jax.experimental.pallas import tpu as pltpu
from jax.experimental.pallas import tpu_sc as plsc

N = 10000
E = 320000
D_IN = 128
HID = 64
D_OUT = 128

NC = 2
NS = 16
NWORK = NC * NS
EPW = E // NWORK
WIN = 125
WPS = EPW // WIN
NBUF = 4
WMAIN = ((WPS - NBUF) // NBUF) * NBUF
N_PAD = 10240
RPS = N_PAD // NS
DEGW = 16
NH = N_PAD // 2
AP = NC * N_PAD * HID // 128
NER = E // 128

_mesh = plsc.VectorSubcoreMesh(core_axis_name="c", subcore_axis_name="s")
_sc_params = pltpu.CompilerParams(use_tc_tiling_on_sc=False)


DPS = NH // NS


@functools.partial(
    pl.kernel,
    out_type=jax.ShapeDtypeStruct((NC * NH, 128), jnp.float32),
    mesh=_mesh,
    scratch_types=[
        pltpu.VMEM((WPS, WIN), jnp.int32),
        pltpu.VMEM((WIN, DEGW), jnp.float32),
        pltpu.VMEM_SHARED((N_PAD, DEGW), jnp.float32),
        pltpu.VMEM((DPS, DEGW), jnp.float32),
        pltpu.VMEM((DPS, DEGW), jnp.float32),
        pltpu.VMEM((DPS, 128), jnp.float32),
    ],
    compiler_params=_sc_params,
)
def _deg_kernel(ei_hbm, vfull_hbm, zeros_hbm, out_hbm, dst_v, vfull, acc,
                st_lo, st_hi, st_out):
    c = lax.axis_index("c")
    s = lax.axis_index("s")
    wid = c * NS + s
    r0 = s * RPS
    pltpu.sync_copy(zeros_hbm.at[pl.ds(r0, RPS)], acc.at[pl.ds(r0, RPS)])
    pltpu.sync_copy(ei_hbm.at[1, wid], dst_v)
    pltpu.sync_copy(vfull_hbm, vfull)
    plsc.subcore_barrier()

    @pl.loop(0, WPS)
    def _(w):
        pltpu.sync_copy(vfull, acc.at[dst_v.at[w]], add=True)

    plsc.subcore_barrier()
    pltpu.sync_copy(acc.at[pl.ds(s * DPS, DPS)], st_lo)
    pltpu.sync_copy(acc.at[pl.ds(NH + s * DPS, DPS)], st_hi)

    @pl.loop(0, DPS)
    def _(r):
        lo = st_lo.at[r][...]
        hi = st_hi.at[r][...]
        for k in range(4):
            st_out.at[r, pl.ds(16 * k, 16)][...] = lo
            st_out.at[r, pl.ds(64 + 16 * k, 16)][...] = hi

    pltpu.sync_copy(st_out, out_hbm.at[pl.ds(c * NH + s * DPS, DPS)])


@functools.partial(
    pl.kernel,
    out_type=jax.ShapeDtypeStruct((NC * N_PAD, HID), jnp.float32),
    mesh=_mesh,
    scratch_types=[
        pltpu.VMEM((WPS, WIN), jnp.int32),
        pltpu.VMEM((WPS, WIN), jnp.int32),
    ] + [pltpu.VMEM((WIN, HID), jnp.float32) for _ in range(NBUF)] + [
        pltpu.VMEM_SHARED((N_PAD, HID), jnp.float32),
    ] + [pltpu.SemaphoreType.DMA for _ in range(NBUF)],
    compiler_params=_sc_params,
)
def _msg_kernel(hs_hbm, srcw_hbm, dstw_hbm, zeros_hbm, out_hbm,
                src_v, dst_v, *rest):
    bufs = rest[:NBUF]
    acc = rest[NBUF]
    gsem = rest[NBUF + 1:]
    c = lax.axis_index("c")
    s = lax.axis_index("s")
    wid = c * NS + s
    r0 = s * RPS
    pltpu.sync_copy(zeros_hbm.at[pl.ds(r0, RPS)], acc.at[pl.ds(r0, RPS)])
    pltpu.sync_copy(srcw_hbm.at[wid], src_v)
    pltpu.sync_copy(dstw_hbm.at[wid], dst_v)
    plsc.subcore_barrier()

    def _wait_gather(i):
        pltpu.make_async_copy(hs_hbm.at[pl.ds(0, WIN)], bufs[i], gsem[i]).wait()

    def _slot(w, i, issue):
        _wait_gather(i)
        pltpu.sync_copy(bufs[i], acc.at[dst_v.at[w]], add=True)
        if issue:
            pltpu.async_copy(hs_hbm.at[src_v.at[w + NBUF]], bufs[i], gsem[i])

    for i in range(NBUF):
        pltpu.async_copy(hs_hbm.at[src_v.at[i]], bufs[i], gsem[i])

    @pl.loop(0, WMAIN, step=NBUF)
    def _(w):
        for i in range(NBUF):
            _wait_gather(i)
            pltpu.sync_copy(bufs[i], acc.at[dst_v.at[w + i]], add=True)
            pltpu.async_copy(hs_hbm.at[src_v.at[w + NBUF + i]], bufs[i],
                             gsem[i])

    for w in range(WMAIN, WPS):
        _slot(w, w % NBUF, w + NBUF < WPS)
    plsc.subcore_barrier()
    pltpu.sync_copy(acc.at[pl.ds(r0, RPS)],
                    out_hbm.at[pl.ds(c * N_PAD + r0, RPS)])


def _t0_body(x_ref, w1_ref, ei_ref, hp_ref, srct_ref, dstt_ref):
    h = jnp.dot(x_ref[...], w1_ref[...], preferred_element_type=jnp.float32)
    lo = h[0:NH, :]
    hi = jnp.concatenate(
        [h[NH:N, :], jnp.zeros((N_PAD - N, HID), jnp.float32)], axis=0)
    hp_ref[...] = jnp.concatenate([lo, hi], axis=1)
    s = ei_ref[0]
    srct_ref[...] = 2 * s - (2 * NH - 1) * (s >= NH).astype(jnp.int32)
    d = ei_ref[1]
    dstt_ref[...] = 2 * d - (2 * NH - 1) * (d >= NH).astype(jnp.int32)


def _t1_body(hp_ref, degp_ref, hsp_ref, dinvp_ref):
    dinv_p = lax.rsqrt(degp_ref[0:NH, :] + degp_ref[NH:2 * NH, :] + 1.0)
    dinvp_ref[...] = dinv_p
    hsp_ref[...] = hp_ref[...] * dinv_p


def _t2_body(aggp_ref, hsp_ref, dinvp_ref, b1p_ref, h2sp_ref):
    dinv_p = dinvp_ref[...]
    out1 = dinv_p * (aggp_ref[0:NH, :] + aggp_ref[NH:2 * NH, :]
                     + hsp_ref[...]) + b1p_ref[...]
    r = jnp.maximum(out1, 0.0)
    s = r * r
    i0 = 1.0 / jnp.maximum(
        jnp.sqrt(jnp.sum(s[:, 0:HID], axis=1, keepdims=True)), 1e-12)
    i1 = 1.0 / jnp.maximum(
        jnp.sqrt(jnp.sum(s[:, HID:128], axis=1, keepdims=True)), 1e-12)
    scale = jnp.concatenate([jnp.broadcast_to(i0, (NH, HID)),
                             jnp.broadcast_to(i1, (NH, HID))], axis=1)
    h2sp_ref[...] = r * scale * dinv_p


def _t3_body(aggp_ref, h2sp_ref, dinvp_ref, w2_ref, b2_ref, out_ref):
    a0 = aggp_ref[0:NH, :]
    a1 = aggp_ref[NH:2 * NH, :]
    pre = dinvp_ref[...] * (a0 + a1 + h2sp_ref[...])
    o_lo = jnp.dot(pre[:, 0:HID], w2_ref[...],
                   preferred_element_type=jnp.float32)
    o_hi = jnp.dot(pre[:, HID:128], w2_ref[...],
                   preferred_element_type=jnp.float32)
    out_ref[...] = (
        jnp.concatenate([o_lo, o_hi[0:N - NH, :]], axis=0) + b2_ref[...])


_f32 = jnp.float32


def kernel(x, edge_index, W1, b1, W2, b2):
    ei_r = edge_index.reshape(2, NER, 128)
    ei_sc = ei_r.reshape(2, NWORK, WPS, WIN)
    zeros_deg = jnp.zeros((N_PAD, DEGW), _f32)
    zeros_hid = jnp.zeros((N_PAD, HID), _f32)
    vfull = jnp.ones((WIN, DEGW), _f32)
    b1p = jnp.concatenate([b1, b1]).reshape(1, 128)

    degp2 = _deg_kernel(ei_sc, vfull, zeros_deg)
    hp, srct, dstt = pl.pallas_call(
        _t0_body,
        out_shape=(jax.ShapeDtypeStruct((NH, 128), _f32),
                   jax.ShapeDtypeStruct((NER, 128), jnp.int32),
                   jax.ShapeDtypeStruct((NER, 128), jnp.int32)),
    )(x, W1, ei_r)
    srcw = srct.reshape(NWORK, WPS, WIN)
    dstw = dstt.reshape(NWORK, WPS, WIN)

    hsp, dinvp = pl.pallas_call(
        _t1_body,
        out_shape=(jax.ShapeDtypeStruct((NH, 128), _f32),
                   jax.ShapeDtypeStruct((NH, 128), _f32)),
    )(hp, degp2)

    agg1p = _msg_kernel(hsp.reshape(N_PAD, HID), srcw, dstw,
                        zeros_hid).reshape(AP, 128)

    h2sp = pl.pallas_call(
        _t2_body,
        out_shape=jax.ShapeDtypeStruct((NH, 128), _f32),
    )(agg1p, hsp, dinvp, b1p)

    agg2p = _msg_kernel(h2sp.reshape(N_PAD, HID), srcw, dstw,
                        zeros_hid).reshape(AP, 128)

    out = pl.pallas_call(
        _t3_body,
        out_shape=jax.ShapeDtypeStruct((N, D_OUT), _f32),
    )(agg2p, h2sp, dinvp, W2, b2.reshape(1, D_OUT))

    return out
